# baseline (speedup 1.0000x reference)
import functools

import jax
import jax.numpy as jnp
from jax import lax
from jax.experimental import pallas as pl
from jax.experimental.pallas import tpu as pltpu
from jax.experimental.pallas import tpu_sc as plsc

HIDDEN = 128
MLP_HID = 100
NUM_GRAPHS = 512
N_NODES = 100000
CHUNK = 128
RB = 800
TC_BLOCKS = 41
SC_ROWS = N_NODES - TC_BLOCKS * RB
TC_BLOCK0 = SC_ROWS // RB

_info = plsc.get_sparse_core_info()
NC = _info.num_cores
NS = _info.num_subcores
NW = NC * NS

N_FULL = SC_ROWS // CHUNK
N_CHUNK_PAD = 784
Q, R = divmod(N_FULL, NW)
MAXJ = Q + 1
IWIN = ((MAXJ + 7) // 8 + 1) * 8
SEG_PER_SUB = NUM_GRAPHS // NS


def _seg_sum_sc(x, batch2d, zeros):
    mesh = plsc.VectorSubcoreMesh(core_axis_name="c", subcore_axis_name="s")

    @functools.partial(
        pl.kernel,
        mesh=mesh,
        out_type=jax.ShapeDtypeStruct((NC, NUM_GRAPHS, HIDDEN), jnp.float32),
        scratch_types=[
            pltpu.VMEM((3, CHUNK, HIDDEN), jnp.float32),
            pltpu.VMEM((IWIN, CHUNK), jnp.int32),
            pltpu.VMEM_SHARED((NUM_GRAPHS, HIDDEN), jnp.float32),
            pltpu.SemaphoreType.DMA,
            pltpu.SemaphoreType.DMA,
            pltpu.SemaphoreType.DMA,
        ],
    )
    def seg_sum(x_hbm, b2d_hbm, z_hbm, out_hbm,
                xbuf, ibuf, acc, gsem, isem, ssem):
        cid = lax.axis_index("c")
        sid = lax.axis_index("s")
        wid = sid * NC + cid

        n_my = Q + jnp.where(wid < R, 1, 0)
        chunk0 = wid * Q + jnp.minimum(wid, R)
        istart = (chunk0 // 8) * 8
        ioff = chunk0 - istart

        idesc = pltpu.make_async_copy(
            b2d_hbm.at[pl.ds(istart, IWIN), :], ibuf, isem)
        idesc.start()

        pltpu.sync_copy(z_hbm.at[pl.ds(sid * SEG_PER_SUB, SEG_PER_SUB), :],
                        acc.at[pl.ds(sid * SEG_PER_SUB, SEG_PER_SUB), :])
        plsc.subcore_barrier()

        def gather(t, slot):
            return pltpu.make_async_copy(
                x_hbm.at[pl.ds((chunk0 + t) * CHUNK, CHUNK), :],
                xbuf.at[slot], gsem)

        def scatter_start(t, slot):
            pltpu.async_copy(xbuf.at[slot], acc.at[ibuf.at[ioff + t]],
                             ssem, add=True)

        def scatter_wait(t, slot):
            pltpu.make_async_copy(
                xbuf.at[slot], acc.at[ibuf.at[ioff + t]], ssem).wait()

        gather(0, 0).start()
        idesc.wait()

        @pl.when(n_my > 1)
        def _prime():
            gather(1, 1).start()

        def body(t, carry):
            slot = lax.rem(t, 3)
            gather(t, slot).wait()
            scatter_start(t, slot)

            @pl.when(t >= 1)
            def _drain():
                scatter_wait(t - 1, lax.rem(t - 1, 3))

            @pl.when(t + 2 < n_my)
            def _prefetch():
                gather(t + 2, lax.rem(t + 2, 3)).start()

            return carry

        lax.fori_loop(0, n_my, body, 0)
        scatter_wait(n_my - 1, lax.rem(n_my - 1, 3))

        plsc.subcore_barrier()

        pltpu.sync_copy(acc.at[pl.ds(sid * SEG_PER_SUB, SEG_PER_SUB), :],
                        out_hbm.at[cid, pl.ds(sid * SEG_PER_SUB, SEG_PER_SUB), :])

    return seg_sum(x, batch2d, zeros)


def _tc_seg_body(ids_ref, x_ref, o_ref):
    b = pl.program_id(0)
    ids = ids_ref[0, 0]
    oh = (ids[:, None] == lax.broadcasted_iota(jnp.int32, (RB, NUM_GRAPHS), 1)
          ).astype(jnp.float32)
    p = lax.dot_general(oh, x_ref[...], (((0,), (0,)), ((), ())),
                        preferred_element_type=jnp.float32)

    @pl.when(b == 0)
    def _init():
        o_ref[...] = p

    @pl.when(b > 0)
    def _accum():
        o_ref[...] += p


def _seg_sum_tc(x, batch3d):
    return pl.pallas_call(
        _tc_seg_body,
        grid=(TC_BLOCKS,),
        in_specs=[
            pl.BlockSpec((1, 1, RB), lambda b: (TC_BLOCK0 + b, 0, 0)),
            pl.BlockSpec((RB, HIDDEN), lambda b: (TC_BLOCK0 + b, 0)),
        ],
        out_specs=pl.BlockSpec((NUM_GRAPHS, HIDDEN), lambda b: (0, 0)),
        out_shape=jax.ShapeDtypeStruct((NUM_GRAPHS, HIDDEN), jnp.float32),
    )(batch3d, x)


def _mlp_body(p_ref, ptc_ref, u_ref, w1a, w1b, b1, w2, b2, w3, b3, w4, b4,
              g, bt, o_ref):
    agg = p_ref[0] + p_ref[1] + ptc_ref[...]
    u = u_ref[...]
    f32 = jnp.float32
    h = (jnp.dot(u, w1a[...], preferred_element_type=f32)
         + jnp.dot(agg, w1b[...], preferred_element_type=f32) + b1[...])
    h = jnp.maximum(h, 0.0)
    h = jnp.maximum(jnp.dot(h, w2[...], preferred_element_type=f32) + b2[...], 0.0)
    h = jnp.maximum(jnp.dot(h, w3[...], preferred_element_type=f32) + b3[...], 0.0)
    h = jnp.dot(h, w4[...], preferred_element_type=f32) + b4[...]
    mu = jnp.mean(h, axis=-1, keepdims=True)
    var = jnp.mean((h - mu) ** 2, axis=-1, keepdims=True)
    h = (h - mu) / jnp.sqrt(var + 1e-5) * g[...] + bt[...]
    o_ref[...] = u + h


def kernel(x, edge_index, edge_attr, u, batch,
           W1, b1, W2, b2, W3, b3, W4, b4, gamma, beta):
    del edge_index, edge_attr
    batch_i32 = batch.astype(jnp.int32)
    batch2d = jnp.pad(batch_i32, (0, N_CHUNK_PAD * CHUNK - N_NODES)
                      ).reshape(N_CHUNK_PAD, CHUNK)
    batch3d = batch_i32.reshape(N_NODES // RB, 1, RB)
    zeros = jnp.zeros((NUM_GRAPHS, HIDDEN), jnp.float32)

    partials = _seg_sum_sc(x, batch2d, zeros)
    partial_tc = _seg_sum_tc(x, batch3d)

    out = pl.pallas_call(
        _mlp_body,
        out_shape=jax.ShapeDtypeStruct((NUM_GRAPHS, HIDDEN), jnp.float32),
    )(partials, partial_tc, u,
      W1[:HIDDEN], W1[HIDDEN:], b1.reshape(1, -1),
      W2, b2.reshape(1, -1), W3, b3.reshape(1, -1),
      W4, b4.reshape(1, -1), gamma.reshape(1, -1), beta.reshape(1, -1))
    return out


# speedup vs baseline: 1.0145x; 1.0145x over previous
import functools

import jax
import jax.numpy as jnp
from jax import lax
from jax.experimental import pallas as pl
from jax.experimental.pallas import tpu as pltpu
from jax.experimental.pallas import tpu_sc as plsc

HIDDEN = 128
MLP_HID = 100
NUM_GRAPHS = 512
N_NODES = 100000
CHUNK = 128
RB = 800
TC_BLOCKS = 41
SC_ROWS = N_NODES - TC_BLOCKS * RB
TC_BLOCK0 = SC_ROWS // RB

_info = plsc.get_sparse_core_info()
NC = _info.num_cores
NS = _info.num_subcores
NW = NC * NS

N_FULL = SC_ROWS // CHUNK
Q, R = divmod(N_FULL, NW)
SEG_PER_SUB = NUM_GRAPHS // NS
LANES = 16


def _seg_sum_sc(x, batch_i32):
    mesh = plsc.VectorSubcoreMesh(core_axis_name="c", subcore_axis_name="s")

    @functools.partial(
        pl.kernel,
        mesh=mesh,
        out_type=jax.ShapeDtypeStruct((NC, NUM_GRAPHS, HIDDEN), jnp.float32),
        scratch_types=[
            pltpu.VMEM((3, CHUNK, HIDDEN), jnp.float32),
            pltpu.VMEM((3, CHUNK), jnp.int32),
            pltpu.VMEM((SEG_PER_SUB, HIDDEN), jnp.float32),
            pltpu.VMEM_SHARED((NUM_GRAPHS, HIDDEN), jnp.float32),
            pltpu.SemaphoreType.DMA,
            pltpu.SemaphoreType.DMA,
            pltpu.SemaphoreType.DMA,
        ],
    )
    def seg_sum(x_hbm, b_hbm, out_hbm, xbuf, ibuf, zbuf, acc,
                gsem, isem, ssem):
        cid = lax.axis_index("c")
        sid = lax.axis_index("s")
        wid = sid * NC + cid

        n_my = Q + jnp.where(wid < R, 1, 0)
        chunk0 = wid * Q + jnp.minimum(wid, R)

        def gather(t, slot):
            base = (chunk0 + t) * CHUNK
            return (
                pltpu.make_async_copy(
                    x_hbm.at[pl.ds(base, CHUNK), :], xbuf.at[slot], gsem),
                pltpu.make_async_copy(
                    b_hbm.at[pl.ds(base, CHUNK)], ibuf.at[slot], isem),
            )

        def gather_start(t, slot):
            for d in gather(t, slot):
                d.start()

        def gather_wait(t, slot):
            for d in gather(t, slot):
                d.wait()

        def scatter_start(t, slot):
            pltpu.async_copy(xbuf.at[slot], acc.at[ibuf.at[slot]],
                             ssem, add=True)

        def scatter_wait(t, slot):
            pltpu.make_async_copy(
                xbuf.at[slot], acc.at[ibuf.at[slot]], ssem).wait()

        gather_start(0, 0)

        zv = jnp.zeros((LANES,), jnp.float32)
        for r in range(SEG_PER_SUB):
            for c in range(HIDDEN // LANES):
                zbuf[r, pl.ds(c * LANES, LANES)] = zv
        pltpu.sync_copy(zbuf, acc.at[pl.ds(sid * SEG_PER_SUB, SEG_PER_SUB), :])
        plsc.subcore_barrier()

        @pl.when(n_my > 1)
        def _prime():
            gather_start(1, 1)

        def body(t, carry):
            slot = lax.rem(t, 3)
            gather_wait(t, slot)
            scatter_start(t, slot)

            @pl.when(t >= 1)
            def _drain():
                scatter_wait(t - 1, lax.rem(t - 1, 3))

            @pl.when(t + 2 < n_my)
            def _prefetch():
                gather_start(t + 2, lax.rem(t + 2, 3))

            return carry

        lax.fori_loop(0, n_my, body, 0)
        scatter_wait(n_my - 1, lax.rem(n_my - 1, 3))

        plsc.subcore_barrier()

        pltpu.sync_copy(acc.at[pl.ds(sid * SEG_PER_SUB, SEG_PER_SUB), :],
                        out_hbm.at[cid, pl.ds(sid * SEG_PER_SUB, SEG_PER_SUB), :])

    return seg_sum(x, batch_i32)


def _tc_seg_body(ids_ref, x_ref, o_ref):
    b = pl.program_id(0)
    ids = ids_ref[0, 0]
    oh = (ids[:, None] == lax.broadcasted_iota(jnp.int32, (RB, NUM_GRAPHS), 1)
          ).astype(jnp.bfloat16)
    p = lax.dot_general(oh, x_ref[...].astype(jnp.bfloat16),
                        (((0,), (0,)), ((), ())),
                        preferred_element_type=jnp.float32)

    @pl.when(b == 0)
    def _init():
        o_ref[...] = p

    @pl.when(b > 0)
    def _accum():
        o_ref[...] += p


def _seg_sum_tc(x, batch3d):
    return pl.pallas_call(
        _tc_seg_body,
        grid=(TC_BLOCKS,),
        in_specs=[
            pl.BlockSpec((1, 1, RB), lambda b: (TC_BLOCK0 + b, 0, 0)),
            pl.BlockSpec((RB, HIDDEN), lambda b: (TC_BLOCK0 + b, 0)),
        ],
        out_specs=pl.BlockSpec((NUM_GRAPHS, HIDDEN), lambda b: (0, 0)),
        out_shape=jax.ShapeDtypeStruct((NUM_GRAPHS, HIDDEN), jnp.float32),
    )(batch3d, x)


def _mlp_body(p_ref, ptc_ref, u_ref, w1a, w1b, b1, w2, b2, w3, b3, w4, b4,
              g, bt, o_ref):
    agg = p_ref[0] + p_ref[1] + ptc_ref[...]
    u = u_ref[...]
    f32 = jnp.float32
    h = (jnp.dot(u, w1a[...], preferred_element_type=f32)
         + jnp.dot(agg, w1b[...], preferred_element_type=f32) + b1[...])
    h = jnp.maximum(h, 0.0)
    h = jnp.maximum(jnp.dot(h, w2[...], preferred_element_type=f32) + b2[...], 0.0)
    h = jnp.maximum(jnp.dot(h, w3[...], preferred_element_type=f32) + b3[...], 0.0)
    h = jnp.dot(h, w4[...], preferred_element_type=f32) + b4[...]
    mu = jnp.mean(h, axis=-1, keepdims=True)
    var = jnp.mean((h - mu) ** 2, axis=-1, keepdims=True)
    h = (h - mu) / jnp.sqrt(var + 1e-5) * g[...] + bt[...]
    o_ref[...] = u + h


def kernel(x, edge_index, edge_attr, u, batch,
           W1, b1, W2, b2, W3, b3, W4, b4, gamma, beta):
    del edge_index, edge_attr
    batch_i32 = batch.astype(jnp.int32)
    batch3d = batch_i32.reshape(N_NODES // RB, 1, RB)

    partials = _seg_sum_sc(x, batch_i32)
    partial_tc = _seg_sum_tc(x, batch3d)

    out = pl.pallas_call(
        _mlp_body,
        out_shape=jax.ShapeDtypeStruct((NUM_GRAPHS, HIDDEN), jnp.float32),
    )(partials, partial_tc, u,
      W1[:HIDDEN], W1[HIDDEN:], b1.reshape(1, -1),
      W2, b2.reshape(1, -1), W3, b3.reshape(1, -1),
      W4, b4.reshape(1, -1), gamma.reshape(1, -1), beta.reshape(1, -1))
    return out


# speedup vs baseline: 1.1631x; 1.1465x over previous
import functools

import jax
import jax.numpy as jnp
from jax import lax
from jax.experimental import pallas as pl
from jax.experimental.pallas import tpu as pltpu
from jax.experimental.pallas import tpu_sc as plsc

HIDDEN = 128
MLP_HID = 100
NUM_GRAPHS = 512
N_NODES = 100000
CHUNK = 128
RB = 800
TC_BLOCKS = 37
SC_ROWS = N_NODES - TC_BLOCKS * RB
TC_BLOCK0 = SC_ROWS // RB

_info = plsc.get_sparse_core_info()
NC = _info.num_cores
NS = _info.num_subcores
NW = NC * NS

N_FULL = SC_ROWS // CHUNK
Q, R = divmod(N_FULL, NW)
SEG_PER_SUB = NUM_GRAPHS // NS
LANES = 16


def _seg_sum_sc(x, batch_i32):
    mesh = plsc.VectorSubcoreMesh(core_axis_name="c", subcore_axis_name="s")

    @functools.partial(
        pl.kernel,
        mesh=mesh,
        out_type=jax.ShapeDtypeStruct((NC, NUM_GRAPHS, HIDDEN), jnp.float32),
        scratch_types=[
            pltpu.VMEM((3, CHUNK, HIDDEN), jnp.float32),
            pltpu.VMEM((3, CHUNK), jnp.int32),
            pltpu.VMEM((SEG_PER_SUB, HIDDEN), jnp.float32),
            pltpu.VMEM_SHARED((NUM_GRAPHS, HIDDEN), jnp.float32),
            pltpu.SemaphoreType.DMA,
            pltpu.SemaphoreType.DMA,
            pltpu.SemaphoreType.DMA,
        ],
    )
    def seg_sum(x_hbm, b_hbm, out_hbm, xbuf, ibuf, zbuf, acc,
                gsem, isem, ssem):
        cid = lax.axis_index("c")
        sid = lax.axis_index("s")
        wid = sid * NC + cid

        n_my = Q + jnp.where(wid < R, 1, 0)
        chunk0 = wid * Q + jnp.minimum(wid, R)

        def gather(t, slot):
            base = (chunk0 + t) * CHUNK
            return (
                pltpu.make_async_copy(
                    x_hbm.at[pl.ds(base, CHUNK), :], xbuf.at[slot], gsem),
                pltpu.make_async_copy(
                    b_hbm.at[pl.ds(base, CHUNK)], ibuf.at[slot], isem),
            )

        def gather_start(t, slot):
            for d in gather(t, slot):
                d.start()

        def gather_wait(t, slot):
            for d in gather(t, slot):
                d.wait()

        def scatter_start(t, slot):
            pltpu.async_copy(xbuf.at[slot], acc.at[ibuf.at[slot]],
                             ssem, add=True)

        def scatter_wait(t, slot):
            pltpu.make_async_copy(
                xbuf.at[slot], acc.at[ibuf.at[slot]], ssem).wait()

        gather_start(0, 0)

        zv = jnp.zeros((LANES,), jnp.float32)
        for r in range(SEG_PER_SUB):
            for c in range(HIDDEN // LANES):
                zbuf[r, pl.ds(c * LANES, LANES)] = zv
        pltpu.sync_copy(zbuf, acc.at[pl.ds(sid * SEG_PER_SUB, SEG_PER_SUB), :])
        plsc.subcore_barrier()

        @pl.when(n_my > 1)
        def _prime():
            gather_start(1, 1)

        def body(t, carry):
            slot = lax.rem(t, 3)
            gather_wait(t, slot)
            scatter_start(t, slot)

            @pl.when(t >= 1)
            def _drain():
                scatter_wait(t - 1, lax.rem(t - 1, 3))

            @pl.when(t + 2 < n_my)
            def _prefetch():
                gather_start(t + 2, lax.rem(t + 2, 3))

            return carry

        lax.fori_loop(0, n_my, body, 0)
        scatter_wait(n_my - 1, lax.rem(n_my - 1, 3))

        plsc.subcore_barrier()

        pltpu.sync_copy(acc.at[pl.ds(sid * SEG_PER_SUB, SEG_PER_SUB), :],
                        out_hbm.at[cid, pl.ds(sid * SEG_PER_SUB, SEG_PER_SUB), :])

    return seg_sum(x, batch_i32)


def _tc_seg_body(ids_ref, x_ref, o_ref):
    b = pl.program_id(0)
    ids = ids_ref[0, 0]
    oh_t = (ids[None, :] == lax.broadcasted_iota(jnp.int32, (NUM_GRAPHS, RB), 0)
            ).astype(jnp.bfloat16)
    p = lax.dot_general(oh_t, x_ref[...].astype(jnp.bfloat16),
                        (((1,), (0,)), ((), ())),
                        preferred_element_type=jnp.float32)

    @pl.when(b == 0)
    def _init():
        o_ref[...] = p

    @pl.when(b > 0)
    def _accum():
        o_ref[...] += p


def _seg_sum_tc(x, batch3d):
    return pl.pallas_call(
        _tc_seg_body,
        grid=(TC_BLOCKS,),
        in_specs=[
            pl.BlockSpec((1, 1, RB), lambda b: (TC_BLOCK0 + b, 0, 0)),
            pl.BlockSpec((RB, HIDDEN), lambda b: (TC_BLOCK0 + b, 0)),
        ],
        out_specs=pl.BlockSpec((NUM_GRAPHS, HIDDEN), lambda b: (0, 0)),
        out_shape=jax.ShapeDtypeStruct((NUM_GRAPHS, HIDDEN), jnp.float32),
    )(batch3d, x)


def _mlp_body(p_ref, ptc_ref, u_ref, w1a, w1b, b1, w2, b2, w3, b3, w4, b4,
              g, bt, o_ref):
    agg = p_ref[0] + p_ref[1] + ptc_ref[...]
    u = u_ref[...]
    f32 = jnp.float32
    h = (jnp.dot(u, w1a[...], preferred_element_type=f32)
         + jnp.dot(agg, w1b[...], preferred_element_type=f32) + b1[...])
    h = jnp.maximum(h, 0.0)
    h = jnp.maximum(jnp.dot(h, w2[...], preferred_element_type=f32) + b2[...], 0.0)
    h = jnp.maximum(jnp.dot(h, w3[...], preferred_element_type=f32) + b3[...], 0.0)
    h = jnp.dot(h, w4[...], preferred_element_type=f32) + b4[...]
    mu = jnp.mean(h, axis=-1, keepdims=True)
    var = jnp.mean((h - mu) ** 2, axis=-1, keepdims=True)
    h = (h - mu) / jnp.sqrt(var + 1e-5) * g[...] + bt[...]
    o_ref[...] = u + h


def kernel(x, edge_index, edge_attr, u, batch,
           W1, b1, W2, b2, W3, b3, W4, b4, gamma, beta):
    del edge_index, edge_attr
    batch_i32 = batch.astype(jnp.int32)
    batch3d = batch_i32.reshape(N_NODES // RB, 1, RB)

    partials = _seg_sum_sc(x, batch_i32)
    partial_tc = _seg_sum_tc(x, batch3d)

    out = pl.pallas_call(
        _mlp_body,
        out_shape=jax.ShapeDtypeStruct((NUM_GRAPHS, HIDDEN), jnp.float32),
    )(partials, partial_tc, u,
      W1[:HIDDEN], W1[HIDDEN:], b1.reshape(1, -1),
      W2, b2.reshape(1, -1), W3, b3.reshape(1, -1),
      W4, b4.reshape(1, -1), gamma.reshape(1, -1), beta.reshape(1, -1))
    return out


# speedup vs baseline: 1.2918x; 1.1106x over previous
import functools

import jax
import jax.numpy as jnp
from jax import lax
from jax.experimental import pallas as pl
from jax.experimental.pallas import tpu as pltpu
from jax.experimental.pallas import tpu_sc as plsc

HIDDEN = 128
MLP_HID = 100
NUM_GRAPHS = 512
N_NODES = 100000
CHUNK = 128
RB = 800
TC_BLOCKS = 13
SC_ROWS = N_NODES - TC_BLOCKS * RB
TC_BLOCK0 = SC_ROWS // RB

_info = plsc.get_sparse_core_info()
NC = _info.num_cores
NS = _info.num_subcores
NW = NC * NS

N_FULL = SC_ROWS // CHUNK
Q, R = divmod(N_FULL, NW)
SEG_PER_SUB = NUM_GRAPHS // NS
LANES = 16


def _seg_sum_sc(x, batch_i32):
    mesh = plsc.VectorSubcoreMesh(core_axis_name="c", subcore_axis_name="s")

    @functools.partial(
        pl.kernel,
        mesh=mesh,
        out_type=jax.ShapeDtypeStruct((NC, NUM_GRAPHS, HIDDEN), jnp.float32),
        scratch_types=[
            pltpu.VMEM((3, CHUNK, HIDDEN), jnp.float32),
            pltpu.VMEM((3, CHUNK), jnp.int32),
            pltpu.VMEM((SEG_PER_SUB, HIDDEN), jnp.float32),
            pltpu.VMEM_SHARED((NUM_GRAPHS, HIDDEN), jnp.float32),
            pltpu.SemaphoreType.DMA,
            pltpu.SemaphoreType.DMA,
            pltpu.SemaphoreType.DMA,
        ],
    )
    def seg_sum(x_hbm, b_hbm, out_hbm, xbuf, ibuf, zbuf, acc,
                gsem, isem, ssem):
        cid = lax.axis_index("c")
        sid = lax.axis_index("s")
        wid = sid * NC + cid

        n_my = Q + jnp.where(wid < R, 1, 0)
        chunk0 = wid * Q + jnp.minimum(wid, R)

        def gather(t, slot):
            base = (chunk0 + t) * CHUNK
            return (
                pltpu.make_async_copy(
                    x_hbm.at[pl.ds(base, CHUNK), :], xbuf.at[slot], gsem),
                pltpu.make_async_copy(
                    b_hbm.at[pl.ds(base, CHUNK)], ibuf.at[slot], isem),
            )

        def gather_start(t, slot):
            for d in gather(t, slot):
                d.start()

        def gather_wait(t, slot):
            for d in gather(t, slot):
                d.wait()

        def scatter_start(t, slot):
            pltpu.async_copy(xbuf.at[slot], acc.at[ibuf.at[slot]],
                             ssem, add=True)

        def scatter_wait(t, slot):
            pltpu.make_async_copy(
                xbuf.at[slot], acc.at[ibuf.at[slot]], ssem).wait()

        gather_start(0, 0)

        zv = jnp.zeros((LANES,), jnp.float32)
        for r in range(SEG_PER_SUB):
            for c in range(HIDDEN // LANES):
                zbuf[r, pl.ds(c * LANES, LANES)] = zv
        pltpu.sync_copy(zbuf, acc.at[pl.ds(sid * SEG_PER_SUB, SEG_PER_SUB), :])
        plsc.subcore_barrier()

        @pl.when(n_my > 1)
        def _prime():
            gather_start(1, 1)

        def body(t, carry):
            slot = lax.rem(t, 3)
            gather_wait(t, slot)
            scatter_start(t, slot)

            @pl.when(t >= 1)
            def _drain():
                scatter_wait(t - 1, lax.rem(t - 1, 3))

            @pl.when(t + 2 < n_my)
            def _prefetch():
                gather_start(t + 2, lax.rem(t + 2, 3))

            return carry

        lax.fori_loop(0, n_my, body, 0)
        scatter_wait(n_my - 1, lax.rem(n_my - 1, 3))

        plsc.subcore_barrier()

        pltpu.sync_copy(acc.at[pl.ds(sid * SEG_PER_SUB, SEG_PER_SUB), :],
                        out_hbm.at[cid, pl.ds(sid * SEG_PER_SUB, SEG_PER_SUB), :])

    return seg_sum(x, batch_i32)


def _tc_seg_body(ids_ref, x_ref, o_ref):
    b = pl.program_id(0)
    ids = ids_ref[0, 0]
    oh_t = (ids[None, :] == lax.broadcasted_iota(jnp.int32, (NUM_GRAPHS, RB), 0)
            ).astype(jnp.bfloat16)
    p = lax.dot_general(oh_t, x_ref[...].astype(jnp.bfloat16),
                        (((1,), (0,)), ((), ())),
                        preferred_element_type=jnp.float32)

    @pl.when(b == 0)
    def _init():
        o_ref[...] = p

    @pl.when(b > 0)
    def _accum():
        o_ref[...] += p


def _seg_sum_tc(x, batch3d):
    return pl.pallas_call(
        _tc_seg_body,
        grid=(TC_BLOCKS,),
        in_specs=[
            pl.BlockSpec((1, 1, RB), lambda b: (TC_BLOCK0 + b, 0, 0)),
            pl.BlockSpec((RB, HIDDEN), lambda b: (TC_BLOCK0 + b, 0)),
        ],
        out_specs=pl.BlockSpec((NUM_GRAPHS, HIDDEN), lambda b: (0, 0)),
        out_shape=jax.ShapeDtypeStruct((NUM_GRAPHS, HIDDEN), jnp.float32),
    )(batch3d, x)


def _mlp_body(p_ref, ptc_ref, u_ref, w1a, w1b, b1, w2, b2, w3, b3, w4, b4,
              g, bt, o_ref):
    agg = p_ref[0] + p_ref[1] + ptc_ref[...]
    u = u_ref[...]
    f32 = jnp.float32
    h = (jnp.dot(u, w1a[...], preferred_element_type=f32)
         + jnp.dot(agg, w1b[...], preferred_element_type=f32) + b1[...])
    h = jnp.maximum(h, 0.0)
    h = jnp.maximum(jnp.dot(h, w2[...], preferred_element_type=f32) + b2[...], 0.0)
    h = jnp.maximum(jnp.dot(h, w3[...], preferred_element_type=f32) + b3[...], 0.0)
    h = jnp.dot(h, w4[...], preferred_element_type=f32) + b4[...]
    mu = jnp.mean(h, axis=-1, keepdims=True)
    var = jnp.mean((h - mu) ** 2, axis=-1, keepdims=True)
    h = (h - mu) / jnp.sqrt(var + 1e-5) * g[...] + bt[...]
    o_ref[...] = u + h


def kernel(x, edge_index, edge_attr, u, batch,
           W1, b1, W2, b2, W3, b3, W4, b4, gamma, beta):
    del edge_index, edge_attr
    batch_i32 = batch.astype(jnp.int32)
    batch3d = batch_i32.reshape(N_NODES // RB, 1, RB)

    partials = _seg_sum_sc(x, batch_i32)
    partial_tc = _seg_sum_tc(x, batch3d)

    out = pl.pallas_call(
        _mlp_body,
        out_shape=jax.ShapeDtypeStruct((NUM_GRAPHS, HIDDEN), jnp.float32),
    )(partials, partial_tc, u,
      W1[:HIDDEN], W1[HIDDEN:], b1.reshape(1, -1),
      W2, b2.reshape(1, -1), W3, b3.reshape(1, -1),
      W4, b4.reshape(1, -1), gamma.reshape(1, -1), beta.reshape(1, -1))
    return out


# speedup vs baseline: 1.3521x; 1.0467x over previous
import functools

import jax
import jax.numpy as jnp
from jax import lax
from jax.experimental import pallas as pl
from jax.experimental.pallas import tpu as pltpu
from jax.experimental.pallas import tpu_sc as plsc

HIDDEN = 128
MLP_HID = 100
NUM_GRAPHS = 512
N_NODES = 100000
CHUNK = 128
RB = 800
TC_BLOCKS = 13
SC_ROWS = N_NODES - TC_BLOCKS * RB
TC_BLOCK0 = SC_ROWS // RB

_info = plsc.get_sparse_core_info()
NC = _info.num_cores
NS = _info.num_subcores
NW = NC * NS

N_FULL = SC_ROWS // CHUNK
Q, R = divmod(N_FULL, NW)
SEG_PER_SUB = NUM_GRAPHS // NS
LANES = 16


def _seg_sum_sc(x, batch_i32):
    mesh = plsc.VectorSubcoreMesh(core_axis_name="c", subcore_axis_name="s")

    @functools.partial(
        pl.kernel,
        mesh=mesh,
        out_type=jax.ShapeDtypeStruct((NC, NUM_GRAPHS, HIDDEN), jnp.float32),
        scratch_types=[
            pltpu.VMEM((3, CHUNK, HIDDEN), jnp.float32),
            pltpu.VMEM((3, CHUNK), jnp.int32),
            pltpu.VMEM((SEG_PER_SUB, HIDDEN), jnp.float32),
            pltpu.VMEM((LANES, HIDDEN), jnp.float32),
            pltpu.VMEM((LANES,), jnp.int32),
            pltpu.VMEM_SHARED((NUM_GRAPHS, HIDDEN), jnp.float32),
            pltpu.SemaphoreType.DMA,
            pltpu.SemaphoreType.DMA,
            pltpu.SemaphoreType.DMA,
        ],
    )
    def seg_sum(x_hbm, b_hbm, out_hbm, xbuf, ibuf, zbuf, srow, sidx, acc,
                gsem, isem, ssem):
        cid = lax.axis_index("c")
        sid = lax.axis_index("s")
        wid = sid * NC + cid

        n_my = Q + jnp.where(wid < R, 1, 0)
        chunk0 = wid * Q + jnp.minimum(wid, R)

        def gather(t, slot):
            base = (chunk0 + t) * CHUNK
            return (
                pltpu.make_async_copy(
                    x_hbm.at[pl.ds(base, CHUNK), :], xbuf.at[slot], gsem),
                pltpu.make_async_copy(
                    b_hbm.at[pl.ds(base, CHUNK)], ibuf.at[slot], isem),
            )

        def gather_start(t, slot):
            for d in gather(t, slot):
                d.start()

        def gather_wait(t, slot):
            for d in gather(t, slot):
                d.wait()

        def scatter_start(t, slot):
            pltpu.async_copy(xbuf.at[slot], acc.at[ibuf.at[slot]],
                             ssem, add=True)

        def scatter_wait(t, slot):
            pltpu.make_async_copy(
                xbuf.at[slot], acc.at[ibuf.at[slot]], ssem).wait()

        gather_start(0, 0)

        zv = jnp.zeros((LANES,), jnp.float32)
        for r in range(SEG_PER_SUB):
            for c in range(HIDDEN // LANES):
                zbuf[r, pl.ds(c * LANES, LANES)] = zv
        for r in range(1, LANES):
            for c in range(HIDDEN // LANES):
                srow[r, pl.ds(c * LANES, LANES)] = zv
        pltpu.sync_copy(zbuf, acc.at[pl.ds(sid * SEG_PER_SUB, SEG_PER_SUB), :])
        plsc.subcore_barrier()

        @pl.when(n_my > 1)
        def _prime():
            gather_start(1, 1)

        def body(t, prev_slow):
            slot = lax.rem(t, 3)
            gather_wait(t, slot)
            first16 = ibuf[slot, pl.ds(0, LANES)]
            last16 = ibuf[slot, pl.ds(CHUNK - LANES, LANES)]
            ndiff = last16[LANES - 1] - first16[0]
            is_slow = jnp.where(ndiff != 0, 1, 0)

            @pl.when(ndiff != 0)
            def _slow():
                scatter_start(t, slot)

            @pl.when(prev_slow == 1)
            def _drain():
                scatter_wait(t - 1, lax.rem(t - 1, 3))

            @pl.when(t + 2 < n_my)
            def _prefetch():
                gather_start(t + 2, lax.rem(t + 2, 3))

            @pl.when(ndiff == 0)
            def _fast():
                def sum_body(r, accs):
                    out = []
                    for c in range(HIDDEN // LANES):
                        a = accs[c]
                        for k in range(8):
                            a = a + xbuf[slot, r * 8 + k,
                                         pl.ds(c * LANES, LANES)]
                        out.append(a)
                    return tuple(out)

                accs = lax.fori_loop(
                    0, CHUNK // 8, sum_body,
                    tuple(jnp.zeros((LANES,), jnp.float32)
                          for _ in range(HIDDEN // LANES)))
                for c in range(HIDDEN // LANES):
                    srow[0, pl.ds(c * LANES, LANES)] = accs[c]
                sidx[...] = first16
                pltpu.sync_copy(srow, acc.at[sidx], add=True)

            return is_slow

        final_slow = lax.fori_loop(0, n_my, body, 0)

        @pl.when(final_slow == 1)
        def _final_drain():
            scatter_wait(n_my - 1, lax.rem(n_my - 1, 3))

        plsc.subcore_barrier()

        pltpu.sync_copy(acc.at[pl.ds(sid * SEG_PER_SUB, SEG_PER_SUB), :],
                        out_hbm.at[cid, pl.ds(sid * SEG_PER_SUB, SEG_PER_SUB), :])

    return seg_sum(x, batch_i32)


def _tc_seg_body(ids_ref, x_ref, o_ref):
    b = pl.program_id(0)
    ids = ids_ref[0, 0]
    oh_t = (ids[None, :] == lax.broadcasted_iota(jnp.int32, (NUM_GRAPHS, RB), 0)
            ).astype(jnp.bfloat16)
    p = lax.dot_general(oh_t, x_ref[...].astype(jnp.bfloat16),
                        (((1,), (0,)), ((), ())),
                        preferred_element_type=jnp.float32)

    @pl.when(b == 0)
    def _init():
        o_ref[...] = p

    @pl.when(b > 0)
    def _accum():
        o_ref[...] += p


def _seg_sum_tc(x, batch3d):
    return pl.pallas_call(
        _tc_seg_body,
        grid=(TC_BLOCKS,),
        in_specs=[
            pl.BlockSpec((1, 1, RB), lambda b: (TC_BLOCK0 + b, 0, 0)),
            pl.BlockSpec((RB, HIDDEN), lambda b: (TC_BLOCK0 + b, 0)),
        ],
        out_specs=pl.BlockSpec((NUM_GRAPHS, HIDDEN), lambda b: (0, 0)),
        out_shape=jax.ShapeDtypeStruct((NUM_GRAPHS, HIDDEN), jnp.float32),
    )(batch3d, x)


def _mlp_body(p_ref, ptc_ref, u_ref, w1a, w1b, b1, w2, b2, w3, b3, w4, b4,
              g, bt, o_ref):
    agg = p_ref[0] + p_ref[1] + ptc_ref[...]
    u = u_ref[...]
    f32 = jnp.float32
    h = (jnp.dot(u, w1a[...], preferred_element_type=f32)
         + jnp.dot(agg, w1b[...], preferred_element_type=f32) + b1[...])
    h = jnp.maximum(h, 0.0)
    h = jnp.maximum(jnp.dot(h, w2[...], preferred_element_type=f32) + b2[...], 0.0)
    h = jnp.maximum(jnp.dot(h, w3[...], preferred_element_type=f32) + b3[...], 0.0)
    h = jnp.dot(h, w4[...], preferred_element_type=f32) + b4[...]
    mu = jnp.mean(h, axis=-1, keepdims=True)
    var = jnp.mean((h - mu) ** 2, axis=-1, keepdims=True)
    h = (h - mu) / jnp.sqrt(var + 1e-5) * g[...] + bt[...]
    o_ref[...] = u + h


def kernel(x, edge_index, edge_attr, u, batch,
           W1, b1, W2, b2, W3, b3, W4, b4, gamma, beta):
    del edge_index, edge_attr
    batch_i32 = batch.astype(jnp.int32)
    batch3d = batch_i32.reshape(N_NODES // RB, 1, RB)

    partials = _seg_sum_sc(x, batch_i32)
    partial_tc = _seg_sum_tc(x, batch3d)

    out = pl.pallas_call(
        _mlp_body,
        out_shape=jax.ShapeDtypeStruct((NUM_GRAPHS, HIDDEN), jnp.float32),
    )(partials, partial_tc, u,
      W1[:HIDDEN], W1[HIDDEN:], b1.reshape(1, -1),
      W2, b2.reshape(1, -1), W3, b3.reshape(1, -1),
      W4, b4.reshape(1, -1), gamma.reshape(1, -1), beta.reshape(1, -1))
    return out
